# baseline (speedup 1.0000x reference)
import functools

import jax
import jax.numpy as jnp
from jax import lax
from jax.experimental import pallas as pl
from jax.experimental.pallas import tpu as pltpu
from jax.experimental.pallas import tpu_sc as plsc

B, G, M, T, D, H, V = 8, 4, 512, 1024, 256, 256, 50000
BG = B * G
N_IDX = BG * M + BG * T


def _sc_gather2(table, cid, rel):
    n0, n1 = cid.shape[0], rel.shape[0]
    d = table.shape[1]
    info = plsc.get_sparse_core_info()
    nc, ns = info.num_cores, info.num_subcores
    nw = nc * ns
    chunk = 128
    nch0 = n0 // (nw * chunk)
    nch1 = n1 // (nw * chunk)
    nbuf = 3
    mesh = plsc.VectorSubcoreMesh(core_axis_name="c", subcore_axis_name="s")

    @functools.partial(
        pl.kernel,
        mesh=mesh,
        out_type=(jax.ShapeDtypeStruct((n0, d), jnp.float32),
                  jax.ShapeDtypeStruct((n1, d), jnp.float32)),
        scratch_types=[
            pltpu.VMEM((nch0, chunk), jnp.int32),
            pltpu.VMEM((nch1, chunk), jnp.int32),
        ]
        + [pltpu.VMEM((chunk, d), jnp.float32) for _ in range(nbuf)]
        + [pltpu.SemaphoreType.DMA for _ in range(nbuf)],
    )
    def k(table_hbm, cid_hbm, rel_hbm, out0_hbm, out1_hbm,
          idx0_v, idx1_v, *bufs_and_sems):
        bufs = bufs_and_sems[:nbuf]
        sems = bufs_and_sems[nbuf:]
        wid = lax.axis_index("s") * nc + lax.axis_index("c")
        pltpu.sync_copy(cid_hbm.at[wid], idx0_v)
        pltpu.sync_copy(rel_hbm.at[wid], idx1_v)
        jobs = ([(idx0_v, i, out0_hbm, nch0) for i in range(nch0)]
                + [(idx1_v, i, out1_hbm, nch1) for i in range(nch1)])

        def fire(j, b):
            idx_v, i, _, _ = jobs[j]
            return pltpu.async_copy(table_hbm.at[idx_v.at[i]], bufs[b],
                                    sems[b])

        copies = {}
        for j in range(nbuf):
            copies[j] = fire(j, j)
        for j in range(len(jobs)):
            b = j % nbuf
            _, i, out_hbm, n_ch = jobs[j]
            copies[j].wait()
            pltpu.sync_copy(bufs[b],
                            out_hbm.at[pl.ds((wid * n_ch + i) * chunk, chunk)])
            if j + nbuf < len(jobs):
                copies[j + nbuf] = fire(j + nbuf, b)

    return k(table, cid.reshape(nw, nch0, chunk), rel.reshape(nw, nch1, chunk))


BPS = 2


def _gcn_body(ch_ref, rh_ref, hd_ref, tl_ref, lbl_ref,
              ws0_ref, wn0_ref, ws1_ref, wn1_ref, wr0_ref,
              wt_h_ref, wt_rf_ref, wt_t_ref, llin_ref,
              triple_ref, cause_ref):
    for s in range(BPS):
        _gcn_one(s, ch_ref, rh_ref, hd_ref, tl_ref, lbl_ref,
                 ws0_ref, wn0_ref, ws1_ref, wn1_ref, wr0_ref,
                 wt_h_ref, wt_rf_ref, wt_t_ref, llin_ref,
                 triple_ref, cause_ref)


def _gcn_one(s, ch_ref, rh_ref, hd_ref, tl_ref, lbl_ref,
             ws0_ref, wn0_ref, ws1_ref, wn1_ref, wr0_ref,
             wt_h_ref, wt_rf_ref, wt_t_ref, llin_ref,
             triple_ref, cause_ref):
    f32 = jnp.float32
    bf16 = jnp.bfloat16
    hd = hd_ref[s, 0, :]
    tl = tl_ref[s, 0, :]
    lbl = lbl_ref[s, 0, :]
    cnt = (lbl != -1).astype(bf16)

    iota_tm = lax.broadcasted_iota(jnp.int32, (T, M), 1)
    iota_mt = lax.broadcasted_iota(jnp.int32, (M, T), 0)
    sh_p = (iota_tm == hd[:, None]).astype(bf16)
    st_p = (iota_tm == tl[:, None]).astype(bf16)
    shm_t = (iota_mt == hd[None, :]).astype(bf16) * cnt[None, :]
    stm_t = (iota_mt == tl[None, :]).astype(bf16) * cnt[None, :]

    dot = functools.partial(jnp.dot, preferred_element_type=f32)
    dotb = dot

    a = dotb(stm_t, sh_p) + dotb(shm_t, st_p)
    cnt_out = jnp.sum(a, axis=1)
    c = jnp.maximum(cnt_out, 1.0)[:, None]
    rh_f = rh_ref[s]
    rh0 = rh_f.astype(bf16)
    r0 = dotb(stm_t + shm_t, rh0)
    a_bf = a.astype(bf16)

    ch = ch_ref[s]
    upd = dotb(a_bf, ch.astype(bf16)) - r0
    ch = jax.nn.relu(dot(ch, ws0_ref[...]) + dot(upd, wn0_ref[...]) / c)
    r1 = dot(r0, wr0_ref[...])
    upd = dotb(a_bf, ch.astype(bf16)) - r1
    ch = jax.nn.relu(dot(ch, ws1_ref[...]) + dot(upd, wn1_ref[...]) / c)

    triple = (dotb(sh_p, dot(ch, wt_h_ref[...]).astype(bf16))
              + dotb(rh0, wt_rf_ref[...].astype(bf16))
              + dotb(st_p, dot(ch, wt_t_ref[...]).astype(bf16)))
    triple_ref[s] = triple

    cause_ref[s] = dot(jnp.sum(triple, axis=0, keepdims=True), llin_ref[...])


def _gcn_call(ch0, rh0, hd3, tl3, lbl3, ws0, wn0, ws1, wn1, wr0,
              wt_h, wt_rf, wt_t, llin):
    full = lambda shp: pl.BlockSpec(shp, lambda b: (0,) * len(shp))
    batch3 = lambda shp: pl.BlockSpec((BPS,) + shp, lambda b: (b, 0, 0))
    return pl.pallas_call(
        _gcn_body,
        grid=(BG // BPS,),
        in_specs=[
            batch3((M, D)), batch3((T, D)),
            batch3((1, T)), batch3((1, T)), batch3((1, T)),
            full((D, D)), full((D, D)), full((D, D)), full((D, D)),
            full((D, D)),
            full((D, D)), full((D, D)), full((D, D)), full((D, D)),
        ],
        out_specs=[batch3((T, D)), batch3((1, D))],
        out_shape=[
            jax.ShapeDtypeStruct((BG, T, D), jnp.float32),
            jax.ShapeDtypeStruct((BG, 1, D), jnp.float32),
        ],
    )(ch0, rh0, hd3, tl3, lbl3, ws0, wn0, ws1, wn1, wr0,
      wt_h, wt_rf, wt_t, llin)


def _fuse_wt_r(wr0, wr1, wt_r):
    def body(a_ref, b_ref, c_ref, o_ref):
        dot = functools.partial(jnp.dot, preferred_element_type=jnp.float32)
        o_ref[...] = dot(a_ref[...], dot(b_ref[...], c_ref[...]))
    return pl.pallas_call(
        body, out_shape=jax.ShapeDtypeStruct((D, D), jnp.float32),
    )(wr0, wr1, wt_r)


def _gru_body(xs_ref, wih_f_ref, whh_f_ref, bih_f_ref, bhh_f_ref,
              wih_b_ref, whh_b_ref, bih_b_ref, bhh_b_ref,
              lc_b_ref, lc_f_ref, out_ref):
    dot = functools.partial(jnp.dot, preferred_element_type=jnp.float32)

    def run(step_ids, wih, whh, bih, bhh):
        h = jnp.zeros((B, H), jnp.float32)
        for g in step_ids:
            xt = xs_ref[g]
            gx = dot(xt, wih[...]) + bih[...]
            gh = dot(h, whh[...]) + bhh[...]
            r = jax.nn.sigmoid(gx[:, 0:H] + gh[:, 0:H])
            z = jax.nn.sigmoid(gx[:, H:2 * H] + gh[:, H:2 * H])
            n = jnp.tanh(gx[:, 2 * H:] + r * gh[:, 2 * H:])
            h = (1.0 - z) * n + z * h
        return h

    h_f = run(range(G), wih_f_ref, whh_f_ref, bih_f_ref, bhh_f_ref)
    h_b = run(range(G - 1, -1, -1), wih_b_ref, whh_b_ref, bih_b_ref, bhh_b_ref)
    out_ref[...] = jnp.tanh(dot(h_b, lc_b_ref[...]) + dot(h_f, lc_f_ref[...]))


def _gru_call(xs, wih_f, whh_f, bih_f, bhh_f, wih_b, whh_b, bih_b, bhh_b,
              lc_b, lc_f):
    return pl.pallas_call(
        _gru_body,
        out_shape=jax.ShapeDtypeStruct((B, H), jnp.float32),
    )(xs, wih_f, whh_f, bih_f, bhh_f, wih_b, whh_b, bih_b, bhh_b, lc_b, lc_f)


def kernel(concept_ids, relation, head, tail, triple_label, emb_table,
           W_s0, W_s1, W_n0, W_n1, W_r0, W_r1, W_triple, L_lin,
           Wih_f, Whh_f, bih_f, bhh_f, Wih_b, Whh_b, bih_b, bhh_b, L_cause):
    cid = concept_ids.reshape(BG * M).astype(jnp.int32)
    rel = relation.reshape(BG * T).astype(jnp.int32)

    ch0_flat, rh0_flat = _sc_gather2(emb_table, cid, rel)
    ch0 = ch0_flat.reshape(BG, M, D)
    rh0 = rh0_flat.reshape(BG, T, D)

    hd3 = head.reshape(BG, 1, T).astype(jnp.int32)
    tl3 = tail.reshape(BG, 1, T).astype(jnp.int32)
    lbl3 = triple_label.reshape(BG, 1, T).astype(jnp.int32)

    wt_h = W_triple[0:D]
    wt_rf = _fuse_wt_r(W_r0, W_r1, W_triple[D:2 * D])
    wt_t = W_triple[2 * D:]

    triple, cause = _gcn_call(ch0, rh0, hd3, tl3, lbl3,
                              W_s0, W_n0, W_s1, W_n1, W_r0,
                              wt_h, wt_rf, wt_t, L_lin)

    xs = cause.reshape(B, G, D).transpose(1, 0, 2)

    encoded = _gru_call(xs, Wih_f.T, Whh_f.T, bih_f.reshape(1, 3 * H),
                        bhh_f.reshape(1, 3 * H), Wih_b.T, Whh_b.T,
                        bih_b.reshape(1, 3 * H), bhh_b.reshape(1, 3 * H),
                        L_cause[0:H], L_cause[H:])

    return (triple, encoded)


# speedup vs baseline: 1.0559x; 1.0559x over previous
import functools

import jax
import jax.numpy as jnp
from jax import lax
from jax.experimental import pallas as pl
from jax.experimental.pallas import tpu as pltpu
from jax.experimental.pallas import tpu_sc as plsc

B, G, M, T, D, H, V = 8, 4, 512, 1024, 256, 256, 50000
BG = B * G
N_IDX = BG * M + BG * T


def _sc_gather2(table, cid, rel):
    n0, n1 = cid.shape[0], rel.shape[0]
    d = table.shape[1]
    info = plsc.get_sparse_core_info()
    nc, ns = info.num_cores, info.num_subcores
    nw = nc * ns
    chunk = 128
    nch0 = n0 // (nw * chunk)
    nch1 = n1 // (nw * chunk)
    nbuf = 3
    mesh = plsc.VectorSubcoreMesh(core_axis_name="c", subcore_axis_name="s")

    @functools.partial(
        pl.kernel,
        mesh=mesh,
        out_type=(jax.ShapeDtypeStruct((n0, d), jnp.float32),
                  jax.ShapeDtypeStruct((n1, d), jnp.float32)),
        scratch_types=[
            pltpu.VMEM((nch0, chunk), jnp.int32),
            pltpu.VMEM((nch1, chunk), jnp.int32),
        ]
        + [pltpu.VMEM((chunk, d), jnp.float32) for _ in range(nbuf)]
        + [pltpu.SemaphoreType.DMA for _ in range(nbuf)],
    )
    def k(table_hbm, cid_hbm, rel_hbm, out0_hbm, out1_hbm,
          idx0_v, idx1_v, *bufs_and_sems):
        bufs = bufs_and_sems[:nbuf]
        sems = bufs_and_sems[nbuf:]
        wid = lax.axis_index("s") * nc + lax.axis_index("c")
        pltpu.sync_copy(cid_hbm.at[wid], idx0_v)
        pltpu.sync_copy(rel_hbm.at[wid], idx1_v)
        jobs = ([(idx0_v, i, out0_hbm, nch0) for i in range(nch0)]
                + [(idx1_v, i, out1_hbm, nch1) for i in range(nch1)])

        def fire(j, b):
            idx_v, i, _, _ = jobs[j]
            return pltpu.async_copy(table_hbm.at[idx_v.at[i]], bufs[b],
                                    sems[b])

        copies = {}
        for j in range(nbuf):
            copies[j] = fire(j, j)
        for j in range(len(jobs)):
            b = j % nbuf
            _, i, out_hbm, n_ch = jobs[j]
            copies[j].wait()
            pltpu.sync_copy(bufs[b],
                            out_hbm.at[pl.ds((wid * n_ch + i) * chunk, chunk)])
            if j + nbuf < len(jobs):
                copies[j + nbuf] = fire(j + nbuf, b)

    return k(table, cid.reshape(nw, nch0, chunk), rel.reshape(nw, nch1, chunk))


BPS = 2


def _abuild_body(hd_ref, tl_ref, lbl_ref, ac_ref, pc_ref):
    bf16 = jnp.bfloat16
    for s in range(BPS):
        hd = hd_ref[s, 0, :]
        tl = tl_ref[s, 0, :]
        lbl = lbl_ref[s, 0, :]
        cnt = (lbl != -1).astype(bf16)
        iota_tm = lax.broadcasted_iota(jnp.int32, (T, M), 1)
        iota_mt = lax.broadcasted_iota(jnp.int32, (M, T), 0)
        sh_p = (iota_tm == hd[:, None]).astype(bf16)
        st_p = (iota_tm == tl[:, None]).astype(bf16)
        shm_t = (iota_mt == hd[None, :]).astype(bf16) * cnt[None, :]
        stm_t = (iota_mt == tl[None, :]).astype(bf16) * cnt[None, :]
        dot = functools.partial(jnp.dot, preferred_element_type=jnp.float32)
        a = dot(stm_t, sh_p) + dot(shm_t, st_p)
        inv = (1.0 / jnp.maximum(jnp.sum(a, axis=1), 1.0)).astype(bf16)
        ac_ref[s] = a.astype(bf16) * inv[:, None]
        pc_ref[s] = (stm_t + shm_t) * inv[:, None]


def _abuild_call(hd3, tl3, lbl3):
    batch3 = lambda shp: pl.BlockSpec((BPS,) + shp, lambda b: (b, 0, 0))
    return pl.pallas_call(
        _abuild_body,
        grid=(BG // BPS,),
        in_specs=[batch3((1, T)), batch3((1, T)), batch3((1, T))],
        out_specs=[batch3((M, M)), batch3((M, T))],
        out_shape=[
            jax.ShapeDtypeStruct((BG, M, M), jnp.bfloat16),
            jax.ShapeDtypeStruct((BG, M, T), jnp.bfloat16),
        ],
    )(hd3, tl3, lbl3)


def _gcn_body(ch_ref, rh_ref, hd_ref, tl_ref, ac_ref, pc_ref,
              ws0_ref, wn0_ref, ws1_ref, wn1_ref, wr0_ref,
              wt_h_ref, wt_rf_ref, wt_t_ref, llin_ref,
              triple_ref, cause_ref):
    for s in range(BPS):
        _gcn_one(s, ch_ref, rh_ref, hd_ref, tl_ref, ac_ref, pc_ref,
                 ws0_ref, wn0_ref, ws1_ref, wn1_ref, wr0_ref,
                 wt_h_ref, wt_rf_ref, wt_t_ref, llin_ref,
                 triple_ref, cause_ref)


def _gcn_one(s, ch_ref, rh_ref, hd_ref, tl_ref, ac_ref, pc_ref,
             ws0_ref, wn0_ref, ws1_ref, wn1_ref, wr0_ref,
             wt_h_ref, wt_rf_ref, wt_t_ref, llin_ref,
             triple_ref, cause_ref):
    f32 = jnp.float32
    bf16 = jnp.bfloat16
    hd = hd_ref[s, 0, :]
    tl = tl_ref[s, 0, :]

    iota_tm = lax.broadcasted_iota(jnp.int32, (T, M), 1)
    sh_p = (iota_tm == hd[:, None]).astype(bf16)
    st_p = (iota_tm == tl[:, None]).astype(bf16)

    dot = functools.partial(jnp.dot, preferred_element_type=f32)
    dotb = dot

    a_c = ac_ref[s]
    p_c = pc_ref[s]
    rh_f = rh_ref[s]
    rh0 = rh_f.astype(bf16)

    ch = ch_ref[s]
    r0s = dotb(p_c, rh0)
    upd = dotb(a_c, ch.astype(bf16)) - r0s
    ch = jax.nn.relu(dot(ch, ws0_ref[...]) + dot(upd, wn0_ref[...]))
    r1s = dot(r0s, wr0_ref[...])
    upd = dotb(a_c, ch.astype(bf16)) - r1s
    ch = jax.nn.relu(dot(ch, ws1_ref[...]) + dot(upd, wn1_ref[...]))

    triple = (dotb(sh_p, dot(ch, wt_h_ref[...]).astype(bf16))
              + dotb(rh0, wt_rf_ref[...].astype(bf16))
              + dotb(st_p, dot(ch, wt_t_ref[...]).astype(bf16)))
    triple_ref[s] = triple
    cause_ref[s] = dot(jnp.sum(triple, axis=0, keepdims=True), llin_ref[...])


def _gcn_call(ch0, rh0, hd3, tl3, ac, pc, ws0, wn0, ws1, wn1, wr0,
              wt_h, wt_rf, wt_t, llin):
    full = lambda shp: pl.BlockSpec(shp, lambda b: (0,) * len(shp))
    batch3 = lambda shp: pl.BlockSpec((BPS,) + shp, lambda b: (b, 0, 0))
    return pl.pallas_call(
        _gcn_body,
        grid=(BG // BPS,),
        in_specs=[
            batch3((M, D)), batch3((T, D)),
            batch3((1, T)), batch3((1, T)),
            batch3((M, M)), batch3((M, T)),
            full((D, D)), full((D, D)), full((D, D)), full((D, D)),
            full((D, D)),
            full((D, D)), full((D, D)), full((D, D)), full((D, D)),
        ],
        out_specs=[batch3((T, D)), batch3((1, D))],
        out_shape=[
            jax.ShapeDtypeStruct((BG, T, D), jnp.float32),
            jax.ShapeDtypeStruct((BG, 1, D), jnp.float32),
        ],
    )(ch0, rh0, hd3, tl3, ac, pc, ws0, wn0, ws1, wn1, wr0,
      wt_h, wt_rf, wt_t, llin)


def _fuse_wt_r(wr0, wr1, wt_r):
    def body(a_ref, b_ref, c_ref, o_ref):
        dot = functools.partial(jnp.dot, preferred_element_type=jnp.float32)
        o_ref[...] = dot(a_ref[...], dot(b_ref[...], c_ref[...]))
    return pl.pallas_call(
        body, out_shape=jax.ShapeDtypeStruct((D, D), jnp.float32),
    )(wr0, wr1, wt_r)


def _gru_body(xs_ref, wih_f_ref, whh_f_ref, bih_f_ref, bhh_f_ref,
              wih_b_ref, whh_b_ref, bih_b_ref, bhh_b_ref,
              lc_b_ref, lc_f_ref, out_ref):
    dot = functools.partial(jnp.dot, preferred_element_type=jnp.float32)

    def run(step_ids, wih, whh, bih, bhh):
        h = jnp.zeros((B, H), jnp.float32)
        for g in step_ids:
            xt = xs_ref[g]
            gx = dot(xt, wih[...]) + bih[...]
            gh = dot(h, whh[...]) + bhh[...]
            r = jax.nn.sigmoid(gx[:, 0:H] + gh[:, 0:H])
            z = jax.nn.sigmoid(gx[:, H:2 * H] + gh[:, H:2 * H])
            n = jnp.tanh(gx[:, 2 * H:] + r * gh[:, 2 * H:])
            h = (1.0 - z) * n + z * h
        return h

    h_f = run(range(G), wih_f_ref, whh_f_ref, bih_f_ref, bhh_f_ref)
    h_b = run(range(G - 1, -1, -1), wih_b_ref, whh_b_ref, bih_b_ref, bhh_b_ref)
    out_ref[...] = jnp.tanh(dot(h_b, lc_b_ref[...]) + dot(h_f, lc_f_ref[...]))


def _gru_call(xs, wih_f, whh_f, bih_f, bhh_f, wih_b, whh_b, bih_b, bhh_b,
              lc_b, lc_f):
    return pl.pallas_call(
        _gru_body,
        out_shape=jax.ShapeDtypeStruct((B, H), jnp.float32),
    )(xs, wih_f, whh_f, bih_f, bhh_f, wih_b, whh_b, bih_b, bhh_b, lc_b, lc_f)


def kernel(concept_ids, relation, head, tail, triple_label, emb_table,
           W_s0, W_s1, W_n0, W_n1, W_r0, W_r1, W_triple, L_lin,
           Wih_f, Whh_f, bih_f, bhh_f, Wih_b, Whh_b, bih_b, bhh_b, L_cause):
    cid = concept_ids.reshape(BG * M).astype(jnp.int32)
    rel = relation.reshape(BG * T).astype(jnp.int32)

    ch0_flat, rh0_flat = _sc_gather2(emb_table, cid, rel)
    ch0 = ch0_flat.reshape(BG, M, D)
    rh0 = rh0_flat.reshape(BG, T, D)

    hd3 = head.reshape(BG, 1, T).astype(jnp.int32)
    tl3 = tail.reshape(BG, 1, T).astype(jnp.int32)
    lbl3 = triple_label.reshape(BG, 1, T).astype(jnp.int32)

    wt_h = W_triple[0:D]
    wt_rf = _fuse_wt_r(W_r0, W_r1, W_triple[D:2 * D])
    wt_t = W_triple[2 * D:]

    ac, pc = _abuild_call(hd3, tl3, lbl3)
    triple, cause = _gcn_call(ch0, rh0, hd3, tl3, ac, pc,
                              W_s0, W_n0, W_s1, W_n1, W_r0,
                              wt_h, wt_rf, wt_t, L_lin)

    xs = cause.reshape(B, G, D).transpose(1, 0, 2)

    encoded = _gru_call(xs, Wih_f.T, Whh_f.T, bih_f.reshape(1, 3 * H),
                        bhh_f.reshape(1, 3 * H), Wih_b.T, Whh_b.T,
                        bih_b.reshape(1, 3 * H), bhh_b.reshape(1, 3 * H),
                        L_cause[0:H], L_cause[H:])

    return (triple, encoded)


# speedup vs baseline: 1.0744x; 1.0176x over previous
import functools

import jax
import jax.numpy as jnp
from jax import lax
from jax.experimental import pallas as pl
from jax.experimental.pallas import tpu as pltpu
from jax.experimental.pallas import tpu_sc as plsc

B, G, M, T, D, H, V = 8, 4, 512, 1024, 256, 256, 50000
BG = B * G
N_IDX = BG * M + BG * T


def _sc_gather2(table, cid, rel):
    n0, n1 = cid.shape[0], rel.shape[0]
    d = table.shape[1]
    info = plsc.get_sparse_core_info()
    nc, ns = info.num_cores, info.num_subcores
    nw = nc * ns
    chunk = 128
    nch0 = n0 // (nw * chunk)
    nch1 = n1 // (nw * chunk)
    nbuf = 3
    mesh = plsc.VectorSubcoreMesh(core_axis_name="c", subcore_axis_name="s")

    @functools.partial(
        pl.kernel,
        mesh=mesh,
        out_type=(jax.ShapeDtypeStruct((n0, d), jnp.float32),
                  jax.ShapeDtypeStruct((n1, d), jnp.float32)),
        scratch_types=[
            pltpu.VMEM((nch0, chunk), jnp.int32),
            pltpu.VMEM((nch1, chunk), jnp.int32),
        ]
        + [pltpu.VMEM((chunk, d), jnp.float32) for _ in range(nbuf)]
        + [pltpu.SemaphoreType.DMA for _ in range(nbuf)],
    )
    def k(table_hbm, cid_hbm, rel_hbm, out0_hbm, out1_hbm,
          idx0_v, idx1_v, *bufs_and_sems):
        bufs = bufs_and_sems[:nbuf]
        sems = bufs_and_sems[nbuf:]
        wid = lax.axis_index("s") * nc + lax.axis_index("c")
        pltpu.sync_copy(cid_hbm.at[wid], idx0_v)
        pltpu.sync_copy(rel_hbm.at[wid], idx1_v)
        jobs = ([(idx0_v, i, out0_hbm, nch0) for i in range(nch0)]
                + [(idx1_v, i, out1_hbm, nch1) for i in range(nch1)])

        def fire(j, b):
            idx_v, i, _, _ = jobs[j]
            return pltpu.async_copy(table_hbm.at[idx_v.at[i]], bufs[b],
                                    sems[b])

        copies = {}
        for j in range(nbuf):
            copies[j] = fire(j, j)
        for j in range(len(jobs)):
            b = j % nbuf
            _, i, out_hbm, n_ch = jobs[j]
            copies[j].wait()
            pltpu.sync_copy(bufs[b],
                            out_hbm.at[pl.ds((wid * n_ch + i) * chunk, chunk)])
            if j + nbuf < len(jobs):
                copies[j + nbuf] = fire(j + nbuf, b)

    return k(table, cid.reshape(nw, nch0, chunk), rel.reshape(nw, nch1, chunk))


BPS = 2


def _abuild_body(hd_ref, tl_ref, lbl_ref, ac_ref, pc_ref):
    bf16 = jnp.bfloat16
    for s in range(BPS):
        hd = hd_ref[s, 0, :]
        tl = tl_ref[s, 0, :]
        lbl = lbl_ref[s, 0, :]
        cnt = (lbl != -1).astype(bf16)
        iota_tm = lax.broadcasted_iota(jnp.int32, (T, M), 1)
        iota_mt = lax.broadcasted_iota(jnp.int32, (M, T), 0)
        sh_p = (iota_tm == hd[:, None]).astype(bf16)
        st_p = (iota_tm == tl[:, None]).astype(bf16)
        shm_t = (iota_mt == hd[None, :]).astype(bf16) * cnt[None, :]
        stm_t = (iota_mt == tl[None, :]).astype(bf16) * cnt[None, :]
        dot = functools.partial(jnp.dot, preferred_element_type=jnp.float32)
        a = dot(stm_t, sh_p) + dot(shm_t, st_p)
        ac_ref[s] = a.astype(bf16)
        pc_ref[s] = stm_t + shm_t


def _abuild_call(hd3, tl3, lbl3):
    batch3 = lambda shp: pl.BlockSpec((BPS,) + shp, lambda b: (b, 0, 0))
    return pl.pallas_call(
        _abuild_body,
        grid=(BG // BPS,),
        in_specs=[batch3((1, T)), batch3((1, T)), batch3((1, T))],
        out_specs=[batch3((M, M)), batch3((M, T))],
        out_shape=[
            jax.ShapeDtypeStruct((BG, M, M), jnp.bfloat16),
            jax.ShapeDtypeStruct((BG, M, T), jnp.bfloat16),
        ],
    )(hd3, tl3, lbl3)


def _gcn_body(ch_ref, rh_ref, hd_ref, tl_ref, ac_ref, pc_ref,
              ws0_ref, wn0_ref, ws1_ref, wn1_ref, wr0_ref,
              wt_h_ref, wt_rf_ref, wt_t_ref, llin_ref,
              triple_ref, cause_ref):
    for s in range(BPS):
        _gcn_one(s, ch_ref, rh_ref, hd_ref, tl_ref, ac_ref, pc_ref,
                 ws0_ref, wn0_ref, ws1_ref, wn1_ref, wr0_ref,
                 wt_h_ref, wt_rf_ref, wt_t_ref, llin_ref,
                 triple_ref, cause_ref)


def _gcn_one(s, ch_ref, rh_ref, hd_ref, tl_ref, ac_ref, pc_ref,
             ws0_ref, wn0_ref, ws1_ref, wn1_ref, wr0_ref,
             wt_h_ref, wt_rf_ref, wt_t_ref, llin_ref,
             triple_ref, cause_ref):
    f32 = jnp.float32
    bf16 = jnp.bfloat16
    hd = hd_ref[s, 0, :]
    tl = tl_ref[s, 0, :]

    iota_tm = lax.broadcasted_iota(jnp.int32, (T, M), 1)
    sh_p = (iota_tm == hd[:, None]).astype(bf16)
    st_p = (iota_tm == tl[:, None]).astype(bf16)

    dot = functools.partial(jnp.dot, preferred_element_type=f32)
    dotb = dot

    a_c = ac_ref[s]
    p_c = pc_ref[s]
    rh_f = rh_ref[s]
    rh0 = rh_f.astype(bf16)

    c = jnp.maximum(jnp.sum(a_c.astype(f32), axis=1), 1.0)[:, None]
    ch = ch_ref[s]
    r0 = dotb(p_c, rh0)
    upd = dotb(a_c, ch.astype(bf16)) - r0
    ch = jax.nn.relu(dot(ch, ws0_ref[...]) + dot(upd, wn0_ref[...]) / c)
    r1 = dot(r0, wr0_ref[...])
    upd = dotb(a_c, ch.astype(bf16)) - r1
    ch = jax.nn.relu(dot(ch, ws1_ref[...]) + dot(upd, wn1_ref[...]) / c)

    triple = (dotb(sh_p, dot(ch, wt_h_ref[...]).astype(bf16))
              + dotb(rh0, wt_rf_ref[...].astype(bf16))
              + dotb(st_p, dot(ch, wt_t_ref[...]).astype(bf16)))
    triple_ref[s] = triple
    cause_ref[s] = dot(jnp.sum(triple, axis=0, keepdims=True), llin_ref[...])


def _gcn_call(ch0, rh0, hd3, tl3, ac, pc, ws0, wn0, ws1, wn1, wr0,
              wt_h, wt_rf, wt_t, llin):
    full = lambda shp: pl.BlockSpec(shp, lambda b: (0,) * len(shp))
    batch3 = lambda shp: pl.BlockSpec((BPS,) + shp, lambda b: (b, 0, 0))
    return pl.pallas_call(
        _gcn_body,
        grid=(BG // BPS,),
        in_specs=[
            batch3((M, D)), batch3((T, D)),
            batch3((1, T)), batch3((1, T)),
            batch3((M, M)), batch3((M, T)),
            full((D, D)), full((D, D)), full((D, D)), full((D, D)),
            full((D, D)),
            full((D, D)), full((D, D)), full((D, D)), full((D, D)),
        ],
        out_specs=[batch3((T, D)), batch3((1, D))],
        out_shape=[
            jax.ShapeDtypeStruct((BG, T, D), jnp.float32),
            jax.ShapeDtypeStruct((BG, 1, D), jnp.float32),
        ],
    )(ch0, rh0, hd3, tl3, ac, pc, ws0, wn0, ws1, wn1, wr0,
      wt_h, wt_rf, wt_t, llin)


def _fuse_wt_r(wr0, wr1, wt_r):
    def body(a_ref, b_ref, c_ref, o_ref):
        dot = functools.partial(jnp.dot, preferred_element_type=jnp.float32)
        o_ref[...] = dot(a_ref[...], dot(b_ref[...], c_ref[...]))
    return pl.pallas_call(
        body, out_shape=jax.ShapeDtypeStruct((D, D), jnp.float32),
    )(wr0, wr1, wt_r)


def _gru_body(xs_ref, wih_f_ref, whh_f_ref, bih_f_ref, bhh_f_ref,
              wih_b_ref, whh_b_ref, bih_b_ref, bhh_b_ref,
              lc_b_ref, lc_f_ref, out_ref):
    dot = functools.partial(jnp.dot, preferred_element_type=jnp.float32)

    def run(step_ids, wih, whh, bih, bhh):
        h = jnp.zeros((B, H), jnp.float32)
        for g in step_ids:
            xt = xs_ref[g]
            gx = dot(xt, wih[...]) + bih[...]
            gh = dot(h, whh[...]) + bhh[...]
            r = jax.nn.sigmoid(gx[:, 0:H] + gh[:, 0:H])
            z = jax.nn.sigmoid(gx[:, H:2 * H] + gh[:, H:2 * H])
            n = jnp.tanh(gx[:, 2 * H:] + r * gh[:, 2 * H:])
            h = (1.0 - z) * n + z * h
        return h

    h_f = run(range(G), wih_f_ref, whh_f_ref, bih_f_ref, bhh_f_ref)
    h_b = run(range(G - 1, -1, -1), wih_b_ref, whh_b_ref, bih_b_ref, bhh_b_ref)
    out_ref[...] = jnp.tanh(dot(h_b, lc_b_ref[...]) + dot(h_f, lc_f_ref[...]))


def _gru_call(xs, wih_f, whh_f, bih_f, bhh_f, wih_b, whh_b, bih_b, bhh_b,
              lc_b, lc_f):
    return pl.pallas_call(
        _gru_body,
        out_shape=jax.ShapeDtypeStruct((B, H), jnp.float32),
    )(xs, wih_f, whh_f, bih_f, bhh_f, wih_b, whh_b, bih_b, bhh_b, lc_b, lc_f)


def kernel(concept_ids, relation, head, tail, triple_label, emb_table,
           W_s0, W_s1, W_n0, W_n1, W_r0, W_r1, W_triple, L_lin,
           Wih_f, Whh_f, bih_f, bhh_f, Wih_b, Whh_b, bih_b, bhh_b, L_cause):
    cid = concept_ids.reshape(BG * M).astype(jnp.int32)
    rel = relation.reshape(BG * T).astype(jnp.int32)

    ch0_flat, rh0_flat = _sc_gather2(emb_table, cid, rel)
    ch0 = ch0_flat.reshape(BG, M, D)
    rh0 = rh0_flat.reshape(BG, T, D)

    hd3 = head.reshape(BG, 1, T).astype(jnp.int32)
    tl3 = tail.reshape(BG, 1, T).astype(jnp.int32)
    lbl3 = triple_label.reshape(BG, 1, T).astype(jnp.int32)

    wt_h = W_triple[0:D]
    wt_rf = _fuse_wt_r(W_r0, W_r1, W_triple[D:2 * D])
    wt_t = W_triple[2 * D:]

    ac, pc = _abuild_call(hd3, tl3, lbl3)
    triple, cause = _gcn_call(ch0, rh0, hd3, tl3, ac, pc,
                              W_s0, W_n0, W_s1, W_n1, W_r0,
                              wt_h, wt_rf, wt_t, L_lin)

    xs = cause.reshape(B, G, D).transpose(1, 0, 2)

    encoded = _gru_call(xs, Wih_f.T, Whh_f.T, bih_f.reshape(1, 3 * H),
                        bhh_f.reshape(1, 3 * H), Wih_b.T, Whh_b.T,
                        bih_b.reshape(1, 3 * H), bhh_b.reshape(1, 3 * H),
                        L_cause[0:H], L_cause[H:])

    return (triple, encoded)


# speedup vs baseline: 1.0777x; 1.0031x over previous
import functools

import jax
import jax.numpy as jnp
from jax import lax
from jax.experimental import pallas as pl
from jax.experimental.pallas import tpu as pltpu
from jax.experimental.pallas import tpu_sc as plsc

B, G, M, T, D, H, V = 8, 4, 512, 1024, 256, 256, 50000
BG = B * G
N_IDX = BG * M + BG * T


def _sc_gather2(table, cid, rel):
    n0, n1 = cid.shape[0], rel.shape[0]
    d = table.shape[1]
    info = plsc.get_sparse_core_info()
    nc, ns = info.num_cores, info.num_subcores
    nw = nc * ns
    chunk = 128
    nch0 = n0 // (nw * chunk)
    nch1 = n1 // (nw * chunk)
    nbuf = 3
    mesh = plsc.VectorSubcoreMesh(core_axis_name="c", subcore_axis_name="s")

    @functools.partial(
        pl.kernel,
        mesh=mesh,
        out_type=(jax.ShapeDtypeStruct((n0, d), jnp.float32),
                  jax.ShapeDtypeStruct((n1, d), jnp.float32)),
        scratch_types=[
            pltpu.VMEM((nch0, chunk), jnp.int32),
            pltpu.VMEM((nch1, chunk), jnp.int32),
        ]
        + [pltpu.VMEM((chunk, d), jnp.float32) for _ in range(nbuf)]
        + [pltpu.SemaphoreType.DMA for _ in range(nbuf)],
    )
    def k(table_hbm, cid_hbm, rel_hbm, out0_hbm, out1_hbm,
          idx0_v, idx1_v, *bufs_and_sems):
        bufs = bufs_and_sems[:nbuf]
        sems = bufs_and_sems[nbuf:]
        wid = lax.axis_index("s") * nc + lax.axis_index("c")
        pltpu.sync_copy(cid_hbm.at[wid], idx0_v)
        pltpu.sync_copy(rel_hbm.at[wid], idx1_v)
        jobs = ([(idx0_v, i, out0_hbm, nch0) for i in range(nch0)]
                + [(idx1_v, i, out1_hbm, nch1) for i in range(nch1)])

        def fire(j, b):
            idx_v, i, _, _ = jobs[j]
            return pltpu.async_copy(table_hbm.at[idx_v.at[i]], bufs[b],
                                    sems[b])

        copies = {}
        for j in range(nbuf):
            copies[j] = fire(j, j)
        for j in range(len(jobs)):
            b = j % nbuf
            _, i, out_hbm, n_ch = jobs[j]
            copies[j].wait()
            pltpu.sync_copy(bufs[b],
                            out_hbm.at[pl.ds((wid * n_ch + i) * chunk, chunk)])
            if j + nbuf < len(jobs):
                copies[j + nbuf] = fire(j + nbuf, b)

    return k(table, cid.reshape(nw, nch0, chunk), rel.reshape(nw, nch1, chunk))


BPS = 2


def _abuild_body(hd_ref, tl_ref, lbl_ref, wr0_ref, wr1_ref, wtr_ref,
                 ac_ref, pc_ref, wtrf_ref):
    bf16 = jnp.bfloat16
    dotf = functools.partial(jnp.dot, preferred_element_type=jnp.float32)

    @pl.when(pl.program_id(0) == 0)
    def _():
        wtrf_ref[...] = dotf(wr0_ref[...], dotf(wr1_ref[...], wtr_ref[...]))

    for s in range(BPS):
        hd = hd_ref[s, 0, :]
        tl = tl_ref[s, 0, :]
        lbl = lbl_ref[s, 0, :]
        cnt = (lbl != -1).astype(bf16)
        iota_tm = lax.broadcasted_iota(jnp.int32, (T, M), 1)
        iota_mt = lax.broadcasted_iota(jnp.int32, (M, T), 0)
        sh_p = (iota_tm == hd[:, None]).astype(bf16)
        st_p = (iota_tm == tl[:, None]).astype(bf16)
        shm_t = (iota_mt == hd[None, :]).astype(bf16) * cnt[None, :]
        stm_t = (iota_mt == tl[None, :]).astype(bf16) * cnt[None, :]
        dot = functools.partial(jnp.dot, preferred_element_type=jnp.float32)
        a = dot(stm_t, sh_p) + dot(shm_t, st_p)
        ac_ref[s] = a.astype(bf16)
        pc_ref[s] = stm_t + shm_t


def _abuild_call(hd3, tl3, lbl3, wr0, wr1, wtr):
    full = lambda shp: pl.BlockSpec(shp, lambda b: (0,) * len(shp))
    batch3 = lambda shp: pl.BlockSpec((BPS,) + shp, lambda b: (b, 0, 0))
    return pl.pallas_call(
        _abuild_body,
        grid=(BG // BPS,),
        in_specs=[batch3((1, T)), batch3((1, T)), batch3((1, T)),
                  full((D, D)), full((D, D)), full((D, D))],
        out_specs=[batch3((M, M)), batch3((M, T)), full((D, D))],
        out_shape=[
            jax.ShapeDtypeStruct((BG, M, M), jnp.bfloat16),
            jax.ShapeDtypeStruct((BG, M, T), jnp.bfloat16),
            jax.ShapeDtypeStruct((D, D), jnp.float32),
        ],
    )(hd3, tl3, lbl3, wr0, wr1, wtr)


def _gcn_body(ch_ref, rh_ref, hd_ref, tl_ref, ac_ref, pc_ref,
              ws0_ref, wn0_ref, ws1_ref, wn1_ref, wr0_ref,
              wt_h_ref, wt_rf_ref, wt_t_ref, llin_ref,
              triple_ref, cause_ref):
    for s in range(BPS):
        _gcn_one(s, ch_ref, rh_ref, hd_ref, tl_ref, ac_ref, pc_ref,
                 ws0_ref, wn0_ref, ws1_ref, wn1_ref, wr0_ref,
                 wt_h_ref, wt_rf_ref, wt_t_ref, llin_ref,
                 triple_ref, cause_ref)


def _gcn_one(s, ch_ref, rh_ref, hd_ref, tl_ref, ac_ref, pc_ref,
             ws0_ref, wn0_ref, ws1_ref, wn1_ref, wr0_ref,
             wt_h_ref, wt_rf_ref, wt_t_ref, llin_ref,
             triple_ref, cause_ref):
    f32 = jnp.float32
    bf16 = jnp.bfloat16
    hd = hd_ref[s, 0, :]
    tl = tl_ref[s, 0, :]

    iota_tm = lax.broadcasted_iota(jnp.int32, (T, M), 1)
    sh_p = (iota_tm == hd[:, None]).astype(bf16)
    st_p = (iota_tm == tl[:, None]).astype(bf16)

    dot = functools.partial(jnp.dot, preferred_element_type=f32)
    dotb = dot

    a_c = ac_ref[s]
    p_c = pc_ref[s]
    rh_f = rh_ref[s]
    rh0 = rh_f.astype(bf16)

    c = jnp.maximum(jnp.sum(a_c.astype(f32), axis=1), 1.0)[:, None]
    ch = ch_ref[s]
    r0 = dotb(p_c, rh0)
    upd = dotb(a_c, ch.astype(bf16)) - r0
    ch = jax.nn.relu(dot(ch, ws0_ref[...]) + dot(upd, wn0_ref[...]) / c)
    r1 = dot(r0, wr0_ref[...])
    upd = dotb(a_c, ch.astype(bf16)) - r1
    ch = jax.nn.relu(dot(ch, ws1_ref[...]) + dot(upd, wn1_ref[...]) / c)

    triple = (dotb(sh_p, dot(ch, wt_h_ref[...]).astype(bf16))
              + dotb(rh0, wt_rf_ref[...].astype(bf16))
              + dotb(st_p, dot(ch, wt_t_ref[...]).astype(bf16)))
    triple_ref[s] = triple
    cause_ref[s] = dot(jnp.sum(triple, axis=0, keepdims=True), llin_ref[...])


def _gcn_call(ch0, rh0, hd3, tl3, ac, pc, ws0, wn0, ws1, wn1, wr0,
              wt_h, wt_rf, wt_t, llin):
    full = lambda shp: pl.BlockSpec(shp, lambda b: (0,) * len(shp))
    batch3 = lambda shp: pl.BlockSpec((BPS,) + shp, lambda b: (b, 0, 0))
    return pl.pallas_call(
        _gcn_body,
        grid=(BG // BPS,),
        in_specs=[
            batch3((M, D)), batch3((T, D)),
            batch3((1, T)), batch3((1, T)),
            batch3((M, M)), batch3((M, T)),
            full((D, D)), full((D, D)), full((D, D)), full((D, D)),
            full((D, D)),
            full((D, D)), full((D, D)), full((D, D)), full((D, D)),
        ],
        out_specs=[batch3((T, D)), batch3((1, D))],
        out_shape=[
            jax.ShapeDtypeStruct((BG, T, D), jnp.float32),
            jax.ShapeDtypeStruct((BG, 1, D), jnp.float32),
        ],
    )(ch0, rh0, hd3, tl3, ac, pc, ws0, wn0, ws1, wn1, wr0,
      wt_h, wt_rf, wt_t, llin)


def _gru_body(xs_ref, wih_f_ref, whh_f_ref, bih_f_ref, bhh_f_ref,
              wih_b_ref, whh_b_ref, bih_b_ref, bhh_b_ref,
              lc_b_ref, lc_f_ref, out_ref):
    dot = functools.partial(jnp.dot, preferred_element_type=jnp.float32)

    def run(step_ids, wih, whh, bih, bhh):
        h = jnp.zeros((B, H), jnp.float32)
        for g in step_ids:
            xt = xs_ref[g]
            gx = dot(xt, wih[...]) + bih[...]
            gh = dot(h, whh[...]) + bhh[...]
            r = jax.nn.sigmoid(gx[:, 0:H] + gh[:, 0:H])
            z = jax.nn.sigmoid(gx[:, H:2 * H] + gh[:, H:2 * H])
            n = jnp.tanh(gx[:, 2 * H:] + r * gh[:, 2 * H:])
            h = (1.0 - z) * n + z * h
        return h

    h_f = run(range(G), wih_f_ref, whh_f_ref, bih_f_ref, bhh_f_ref)
    h_b = run(range(G - 1, -1, -1), wih_b_ref, whh_b_ref, bih_b_ref, bhh_b_ref)
    out_ref[...] = jnp.tanh(dot(h_b, lc_b_ref[...]) + dot(h_f, lc_f_ref[...]))


def _gru_call(xs, wih_f, whh_f, bih_f, bhh_f, wih_b, whh_b, bih_b, bhh_b,
              lc_b, lc_f):
    return pl.pallas_call(
        _gru_body,
        out_shape=jax.ShapeDtypeStruct((B, H), jnp.float32),
    )(xs, wih_f, whh_f, bih_f, bhh_f, wih_b, whh_b, bih_b, bhh_b, lc_b, lc_f)


def kernel(concept_ids, relation, head, tail, triple_label, emb_table,
           W_s0, W_s1, W_n0, W_n1, W_r0, W_r1, W_triple, L_lin,
           Wih_f, Whh_f, bih_f, bhh_f, Wih_b, Whh_b, bih_b, bhh_b, L_cause):
    cid = concept_ids.reshape(BG * M).astype(jnp.int32)
    rel = relation.reshape(BG * T).astype(jnp.int32)

    ch0_flat, rh0_flat = _sc_gather2(emb_table, cid, rel)
    ch0 = ch0_flat.reshape(BG, M, D)
    rh0 = rh0_flat.reshape(BG, T, D)

    hd3 = head.reshape(BG, 1, T).astype(jnp.int32)
    tl3 = tail.reshape(BG, 1, T).astype(jnp.int32)
    lbl3 = triple_label.reshape(BG, 1, T).astype(jnp.int32)

    wt_h = W_triple[0:D]
    wt_t = W_triple[2 * D:]

    ac, pc, wt_rf = _abuild_call(hd3, tl3, lbl3, W_r0, W_r1,
                                 W_triple[D:2 * D])
    triple, cause = _gcn_call(ch0, rh0, hd3, tl3, ac, pc,
                              W_s0, W_n0, W_s1, W_n1, W_r0,
                              wt_h, wt_rf, wt_t, L_lin)

    xs = cause.reshape(B, G, D).transpose(1, 0, 2)

    encoded = _gru_call(xs, Wih_f.T, Whh_f.T, bih_f.reshape(1, 3 * H),
                        bhh_f.reshape(1, 3 * H), Wih_b.T, Whh_b.T,
                        bih_b.reshape(1, 3 * H), bhh_b.reshape(1, 3 * H),
                        L_cause[0:H], L_cause[H:])

    return (triple, encoded)
